# baseline (speedup 1.0000x reference)
import functools
import jax
import jax.numpy as jnp
from jax import lax
from jax.experimental import pallas as pl
from jax.experimental.pallas import tpu as pltpu
from jax.experimental.pallas import tpu_sc as plsc

N_GENES = 20000
N_CELLS = 50000
N_LATENT = 64
N_COMP = 32
N_HID = 32
B_CELLS = 512
B_GENES = 1000
N_FRAG = 200000
WIN_A = -10000.0
WIN_B = 10000.0
AB = WIN_B - WIN_A
SCALE_LB = 2.0 / AB
INV_SQ = 1.0 / (1.0 + 1e-5) ** 0.5
LOG2PI = 1.8378770664093453

GPAD = 1024
FPAD = 204800
FB = 4096
NFB = FPAD // FB
GB = 128
NGB = GPAD // GB


def _sc_info():
    try:
        info = plsc.get_sparse_core_info()
        return info.num_cores, info.num_subcores
    except Exception:
        return 2, 16


def _sc_gather_tables(lw_tab, gs_tab, rb_tab, cl_tab, g_idx, c_idx):
    nc, ns = _sc_info()
    nw = nc * ns
    gpw = GPAD // nw
    cpw = B_CELLS // nw
    mesh = plsc.VectorSubcoreMesh(core_axis_name="c", subcore_axis_name="s")

    @functools.partial(
        pl.kernel, mesh=mesh,
        out_type=[
            jax.ShapeDtypeStruct((GPAD, 1024), jnp.float32),
            jax.ShapeDtypeStruct((GPAD, 128), jnp.float32),
            jax.ShapeDtypeStruct((GPAD, 128), jnp.float32),
            jax.ShapeDtypeStruct((B_CELLS, 128), jnp.float32),
        ],
        scratch_types=[
            pltpu.VMEM((gpw,), jnp.int32),
            pltpu.VMEM((cpw,), jnp.int32),
            pltpu.VMEM((gpw, 1024), jnp.float32),
            pltpu.VMEM((gpw, 128), jnp.float32),
            pltpu.VMEM((gpw, 128), jnp.float32),
            pltpu.VMEM((cpw, 128), jnp.float32),
            pltpu.SemaphoreType.DMA,
        ],
    )
    def k(lw_hbm, gsa_hbm, gsb_hbm, cl_hbm, gi_hbm, ci_hbm,
          lw_out, gsa_out, gsb_out, cl_out,
          gi_v, ci_v, lw_v, gsa_v, gsb_v, cl_v, sem):
        wid = lax.axis_index("s") * nc + lax.axis_index("c")
        gb = wid * gpw
        cb = wid * cpw
        pltpu.sync_copy(gi_hbm.at[pl.ds(gb, gpw)], gi_v)
        pltpu.async_copy(lw_hbm.at[gi_v], lw_v, sem).wait()
        pltpu.sync_copy(lw_v, lw_out.at[pl.ds(gb, gpw)])
        pltpu.async_copy(gsa_hbm.at[gi_v], gsa_v, sem).wait()
        pltpu.sync_copy(gsa_v, gsa_out.at[pl.ds(gb, gpw)])
        pltpu.async_copy(gsb_hbm.at[gi_v], gsb_v, sem).wait()
        pltpu.sync_copy(gsb_v, gsb_out.at[pl.ds(gb, gpw)])
        pltpu.sync_copy(ci_hbm.at[pl.ds(cb, cpw)], ci_v)
        pltpu.async_copy(cl_hbm.at[ci_v], cl_v, sem).wait()
        pltpu.sync_copy(cl_v, cl_out.at[pl.ds(cb, cpw)])

    return k(lw_tab, gs_tab, rb_tab, cl_tab, g_idx, c_idx)


def _sc_gather_frags(logit_tab, mix_tab, cxg_idx, g_idx):
    nc, ns = _sc_info()
    nw = nc * ns
    fpw = FPAD // nw
    chunk = 320
    nch = fpw // chunk
    mesh = plsc.VectorSubcoreMesh(core_axis_name="c", subcore_axis_name="s")

    @functools.partial(
        pl.kernel, mesh=mesh,
        out_type=[
            jax.ShapeDtypeStruct((FPAD, 128), jnp.float32),
            jax.ShapeDtypeStruct((FPAD, 128), jnp.float32),
        ],
        scratch_types=[
            pltpu.VMEM((chunk,), jnp.int32),
            pltpu.VMEM((chunk,), jnp.int32),
            pltpu.VMEM((chunk, 128), jnp.float32),
            pltpu.VMEM((chunk, 128), jnp.float32),
            pltpu.SemaphoreType.DMA,
        ],
    )
    def k(lt_hbm, mt_hbm, ci_hbm, gi_hbm, d_out, m_out, ci_v, gi_v, d_v, m_v, sem):
        wid = lax.axis_index("s") * nc + lax.axis_index("c")
        for c in range(nch):
            base = wid * fpw + c * chunk
            pltpu.sync_copy(ci_hbm.at[pl.ds(base, chunk)], ci_v)
            pltpu.async_copy(lt_hbm.at[ci_v], d_v, sem).wait()
            pltpu.sync_copy(d_v, d_out.at[pl.ds(base, chunk)])
            pltpu.sync_copy(gi_hbm.at[pl.ds(base, chunk)], gi_v)
            pltpu.async_copy(mt_hbm.at[gi_v], m_v, sem).wait()
            pltpu.sync_copy(m_v, m_out.at[pl.ds(base, chunk)])

    return k(logit_tab, mix_tab, cxg_idx, g_idx)


def _t1_dense(cl_sel, lwT, rwT, rb_row, W1, b1r, g1r, be1r, W2, b2r, g2r, be2r):
    def body(cl_ref, lw_ref, rw_ref, rb_ref, w1_ref, b1_ref, g1_ref, be1_ref,
             w2_ref, b2_ref, g2_ref, be2_ref, logit_ref, mu_ref):
        lat = cl_ref[:, 0:64]
        lib = cl_ref[:, 64:65]
        h = jnp.dot(lat, w1_ref[...], preferred_element_type=jnp.float32) + b1_ref[...]
        h = jnp.maximum(h, 0.0) * INV_SQ * g1_ref[...] + be1_ref[...]
        h = jnp.dot(h, w2_ref[...], preferred_element_type=jnp.float32) + b2_ref[...]
        h = jnp.maximum(h, 0.0) * INV_SQ * g2_ref[...] + be2_ref[...]
        logit_ref[...] = jnp.dot(h, lw_ref[...], preferred_element_type=jnp.float32)
        rho = jnp.dot(h, rw_ref[...], preferred_element_type=jnp.float32)
        mu_ref[...] = rb_ref[...] * jnp.exp(rho) * lib

    return pl.pallas_call(
        body,
        grid=(NGB,),
        in_specs=[
            pl.BlockSpec((B_CELLS, 128), lambda g: (0, 0)),
            pl.BlockSpec((32, GB * 32), lambda g: (0, g)),
            pl.BlockSpec((32, GB), lambda g: (0, g)),
            pl.BlockSpec((1, GB), lambda g: (0, g)),
            pl.BlockSpec((N_LATENT, N_HID), lambda g: (0, 0)),
            pl.BlockSpec((1, N_HID), lambda g: (0, 0)),
            pl.BlockSpec((1, N_HID), lambda g: (0, 0)),
            pl.BlockSpec((1, N_HID), lambda g: (0, 0)),
            pl.BlockSpec((N_HID, N_HID), lambda g: (0, 0)),
            pl.BlockSpec((1, N_HID), lambda g: (0, 0)),
            pl.BlockSpec((1, N_HID), lambda g: (0, 0)),
            pl.BlockSpec((1, N_HID), lambda g: (0, 0)),
        ],
        out_specs=[
            pl.BlockSpec((B_CELLS, GB * 32), lambda g: (0, g)),
            pl.BlockSpec((B_CELLS, GB), lambda g: (0, g)),
        ],
        out_shape=[
            jax.ShapeDtypeStruct((B_CELLS, GPAD * 32), jnp.float32),
            jax.ShapeDtypeStruct((B_CELLS, GPAD), jnp.float32),
        ],
    )(cl_sel, lwT, rwT, rb_row, W1, b1r, g1r, be1r, W2, b2r, g2r, be2r)


def _t2_mixture(delta, mix_g, x0c, x1c, selc):
    def body(d_ref, m_ref, x0_ref, x1_ref, sel_ref, out_ref):
        i = pl.program_id(0)
        raw = m_ref[...]
        loc = jax.nn.sigmoid(raw[:, 0:32])
        scale = SCALE_LB + jnp.exp(raw[:, 32:64])
        lsc = jnp.log(scale)
        dw = d_ref[...]
        sel = sel_ref[...]
        d = jnp.where(sel < 0.5, dw[:, 0:32],
            jnp.where(sel < 1.5, dw[:, 32:64],
            jnp.where(sel < 2.5, dw[:, 64:96], dw[:, 96:128])))
        lg = raw[:, 64:96] + d
        m = jnp.max(lg, axis=1, keepdims=True)
        lse = m + jnp.log(jnp.sum(jnp.exp(lg - m), axis=1, keepdims=True))
        log_mix = lg - lse

        def side(xcol):
            x = (xcol - WIN_A) / AB
            z = (x - loc) / scale
            t = log_mix + (-0.5 * z * z - lsc - 0.5 * LOG2PI)
            tm = jnp.max(t, axis=1, keepdims=True)
            return tm + jnp.log(jnp.sum(jnp.exp(t - tm), axis=1, keepdims=True))

        ll = side(x0_ref[...]) + side(x1_ref[...])
        fidx = i * FB + lax.broadcasted_iota(jnp.int32, (FB, 1), 0)
        ll = jnp.where(fidx < N_FRAG, ll, 0.0)

        @pl.when(i == 0)
        def _():
            out_ref[...] = jnp.zeros_like(out_ref)
        out_ref[...] += jnp.sum(ll).reshape(1, 1)

    return pl.pallas_call(
        body,
        grid=(NFB,),
        in_specs=[
            pl.BlockSpec((FB, 128), lambda i: (i, 0)),
            pl.BlockSpec((FB, 128), lambda i: (i, 0)),
            pl.BlockSpec((FB, 1), lambda i: (i, 0)),
            pl.BlockSpec((FB, 1), lambda i: (i, 0)),
            pl.BlockSpec((FB, 1), lambda i: (i, 0)),
        ],
        out_specs=pl.BlockSpec((1, 1), lambda i: (0, 0)),
        out_shape=jax.ShapeDtypeStruct((1, 1), jnp.float32),
    )(delta, mix_g, x0c, x1c, selc)


def _log_factorial(c):
    x = c + 1.0
    xs = jnp.maximum(x, 9.0)
    inv = 1.0 / xs
    stirling = ((xs - 0.5) * jnp.log(xs) - xs + 0.5 * LOG2PI
                + inv / 12.0 - (inv * inv * inv) / 360.0)
    tab = jnp.where(
        c < 0.5, 0.0,
        jnp.where(c < 1.5, 0.0,
        jnp.where(c < 2.5, 0.6931471805599453,
        jnp.where(c < 3.5, 1.791759469228055,
        jnp.where(c < 4.5, 3.1780538303479458,
        jnp.where(c < 5.5, 4.787491742782046,
        jnp.where(c < 6.5, 6.579251212010101,
                  8.525161361065415)))))))
    return jnp.where(c < 7.5, tab, stirling)


def _t3_poisson(cxg_row3, cxg_col, mu):
    def body(row_ref, col_ref, mu_ref, out_ref, cnt_ref):
        i = pl.program_id(0)

        @pl.when(i == 0)
        def _():
            cnt_ref[...] = jnp.zeros_like(cnt_ref)

        idx_row = row_ref[...].reshape(1, FB).astype(jnp.float32)
        cells_row = jnp.floor((idx_row + 0.5) / float(B_GENES))
        lane_f = lax.broadcasted_iota(jnp.int32, (B_CELLS, FB), 0).astype(jnp.float32)
        valid_row = (i * FB + lax.broadcasted_iota(jnp.int32, (B_CELLS, FB), 1)) < N_FRAG
        cell_ohT = jnp.where((lane_f == cells_row) & valid_row, 1.0, 0.0)

        idx_col = col_ref[...].astype(jnp.float32)
        cells_col = jnp.floor((idx_col + 0.5) / float(B_GENES))
        genes_col = idx_col - cells_col * float(B_GENES)
        gl = lax.broadcasted_iota(jnp.int32, (FB, B_GENES), 1).astype(jnp.float32)
        gene_oh = jnp.where(gl == genes_col, 1.0, 0.0)

        cnt_ref[...] += jnp.dot(cell_ohT.astype(jnp.bfloat16),
                                gene_oh.astype(jnp.bfloat16),
                                preferred_element_type=jnp.float32)

        @pl.when(i == NFB - 1)
        def _():
            cnt = cnt_ref[...]
            mu_v = mu_ref[...]
            pois = cnt * jnp.log(mu_v) - mu_v - _log_factorial(cnt)
            out_ref[...] = jnp.sum(pois).reshape(1, 1)

    return pl.pallas_call(
        body,
        grid=(NFB,),
        in_specs=[
            pl.BlockSpec((1, 1, FB), lambda i: (i, 0, 0)),
            pl.BlockSpec((FB, 1), lambda i: (i, 0)),
            pl.BlockSpec((B_CELLS, B_GENES), lambda i: (0, 0)),
        ],
        out_specs=pl.BlockSpec((1, 1), lambda i: (0, 0)),
        out_shape=jax.ShapeDtypeStruct((1, 1), jnp.float32),
        scratch_shapes=[pltpu.VMEM((B_CELLS, B_GENES), jnp.float32)],
    )(cxg_row3, cxg_col, mu)


def kernel(cells_oi, genes_oi, coordinates, local_gene_ix, local_cellxgene_ix,
           cell_latent_space, W1, b1, g1, be1, W2, b2, g2, be2,
           logit_w, rho_w, mloc, mscale, mlogit, libsize, rho_bias):
    f32 = jnp.float32
    i32 = jnp.int32

    lw_tab = logit_w.reshape(N_GENES, N_HID * N_COMP)
    gs_tab = jnp.concatenate([mloc, mscale, mlogit, rho_w], axis=1)
    rb_tab = jnp.concatenate(
        [rho_bias[:, None], jnp.zeros((N_GENES, 127), f32)], axis=1)
    cl_tab = jnp.concatenate(
        [cell_latent_space, libsize[:, None].astype(f32),
         jnp.zeros((N_CELLS, 63), f32)], axis=1)
    g_idx = jnp.concatenate(
        [genes_oi.astype(i32), jnp.zeros((GPAD - B_GENES,), i32)])
    c_idx = cells_oi.astype(i32)

    lw_sel, gs_sel, rb_sel, cl_sel = _sc_gather_tables(
        lw_tab, gs_tab, rb_tab, cl_tab, g_idx, c_idx)

    lwT = (lw_sel.reshape(GPAD, N_HID, N_COMP)
           .transpose(1, 0, 2).reshape(N_HID, GPAD * N_COMP))
    rwT = gs_sel[:, 96:128].T
    rb_row = rb_sel[:, 0].reshape(1, GPAD)
    mix_tab = gs_sel[:B_GENES]

    b1r = b1.reshape(1, N_HID); g1r = g1.reshape(1, N_HID); be1r = be1.reshape(1, N_HID)
    b2r = b2.reshape(1, N_HID); g2r = g2.reshape(1, N_HID); be2r = be2.reshape(1, N_HID)

    logit_flat2, mu_pad = _t1_dense(cl_sel, lwT, rwT, rb_row,
                                    W1, b1r, g1r, be1r, W2, b2r, g2r, be2r)
    logit_tab = logit_flat2.reshape(B_CELLS * GPAD * N_COMP // 128, 128)
    mu = mu_pad[:, :B_GENES]

    pad = FPAD - N_FRAG
    cxg = jnp.concatenate([local_cellxgene_ix.astype(i32), jnp.zeros((pad,), i32)])
    gix = jnp.concatenate([local_gene_ix.astype(i32), jnp.zeros((pad,), i32)])
    cxg2 = cxg + (GPAD - B_GENES) * (cxg // B_GENES)
    idx_wide = cxg2 // 4
    selc = (gix % 4).astype(f32).reshape(FPAD, 1)
    x0c = jnp.concatenate([coordinates[:, 0:1], jnp.zeros((pad, 1), f32)])
    x1c = jnp.concatenate([coordinates[:, 1:2], jnp.zeros((pad, 1), f32)])

    delta, mix_g = _sc_gather_frags(logit_tab, mix_tab, idx_wide, gix)

    ll = _t2_mixture(delta, mix_g, x0c, x1c, selc)

    cxg_row3 = cxg.reshape(NFB, 1, FB)
    cxg_col = cxg.reshape(FPAD, 1)
    pois = _t3_poisson(cxg_row3, cxg_col, mu)

    return -(ll[0, 0] + pois[0, 0])


# speedup vs baseline: 1.0569x; 1.0569x over previous
import functools
import jax
import jax.numpy as jnp
from jax import lax
from jax.experimental import pallas as pl
from jax.experimental.pallas import tpu as pltpu
from jax.experimental.pallas import tpu_sc as plsc

N_GENES = 20000
N_CELLS = 50000
N_LATENT = 64
N_COMP = 32
N_HID = 32
B_CELLS = 512
B_GENES = 1000
N_FRAG = 200000
WIN_A = -10000.0
WIN_B = 10000.0
AB = WIN_B - WIN_A
SCALE_LB = 2.0 / AB
INV_SQ = 1.0 / (1.0 + 1e-5) ** 0.5
LOG2PI = 1.8378770664093453

GPAD = 1024
FPAD = 204800
FB = 4096
NFB = FPAD // FB
GB = 128
NGB = GPAD // GB


def _sc_info():
    try:
        info = plsc.get_sparse_core_info()
        return info.num_cores, info.num_subcores
    except Exception:
        return 2, 16


def _sc_gather_tables(lw_tab, gs_tab, rb_tab, cl_tab, g_idx, c_idx):
    nc, ns = _sc_info()
    nw = nc * ns
    gpw = GPAD // nw
    cpw = B_CELLS // nw
    mesh = plsc.VectorSubcoreMesh(core_axis_name="c", subcore_axis_name="s")

    @functools.partial(
        pl.kernel, mesh=mesh,
        out_type=[
            jax.ShapeDtypeStruct((GPAD, 1024), jnp.float32),
            jax.ShapeDtypeStruct((GPAD, 128), jnp.float32),
            jax.ShapeDtypeStruct((GPAD, 128), jnp.float32),
            jax.ShapeDtypeStruct((B_CELLS, 128), jnp.float32),
        ],
        scratch_types=[
            pltpu.VMEM((gpw,), jnp.int32),
            pltpu.VMEM((cpw,), jnp.int32),
            pltpu.VMEM((gpw, 1024), jnp.float32),
            pltpu.VMEM((gpw, 128), jnp.float32),
            pltpu.VMEM((gpw, 128), jnp.float32),
            pltpu.VMEM((cpw, 128), jnp.float32),
            pltpu.SemaphoreType.DMA,
        ],
    )
    def k(lw_hbm, gsa_hbm, gsb_hbm, cl_hbm, gi_hbm, ci_hbm,
          lw_out, gsa_out, gsb_out, cl_out,
          gi_v, ci_v, lw_v, gsa_v, gsb_v, cl_v, sem):
        wid = lax.axis_index("s") * nc + lax.axis_index("c")
        gb = wid * gpw
        cb = wid * cpw
        pltpu.sync_copy(gi_hbm.at[pl.ds(gb, gpw)], gi_v)
        pltpu.async_copy(lw_hbm.at[gi_v], lw_v, sem).wait()
        pltpu.sync_copy(lw_v, lw_out.at[pl.ds(gb, gpw)])
        pltpu.async_copy(gsa_hbm.at[gi_v], gsa_v, sem).wait()
        pltpu.sync_copy(gsa_v, gsa_out.at[pl.ds(gb, gpw)])
        pltpu.async_copy(gsb_hbm.at[gi_v], gsb_v, sem).wait()
        pltpu.sync_copy(gsb_v, gsb_out.at[pl.ds(gb, gpw)])
        pltpu.sync_copy(ci_hbm.at[pl.ds(cb, cpw)], ci_v)
        pltpu.async_copy(cl_hbm.at[ci_v], cl_v, sem).wait()
        pltpu.sync_copy(cl_v, cl_out.at[pl.ds(cb, cpw)])

    return k(lw_tab, gs_tab, rb_tab, cl_tab, g_idx, c_idx)


def _sc_gather_frag_rows(tab, idx, label_rows):
    nc, ns = _sc_info()
    nw = nc * ns
    fpw = FPAD // nw
    chunk = 320
    nch = fpw // chunk
    mesh = plsc.VectorSubcoreMesh(core_axis_name="c", subcore_axis_name="s")

    @functools.partial(
        pl.kernel, mesh=mesh,
        out_type=jax.ShapeDtypeStruct((FPAD, 128), jnp.float32),
        scratch_types=[
            pltpu.VMEM((fpw,), jnp.int32),
            pltpu.VMEM((chunk, 128), jnp.float32),
            pltpu.VMEM((chunk, 128), jnp.float32),
            pltpu.SemaphoreType.DMA,
            pltpu.SemaphoreType.DMA,
        ],
    )
    def k(tab_hbm, idx_hbm, out_hbm, idx_v, buf0, buf1, sem0, sem1):
        wid = lax.axis_index("s") * nc + lax.axis_index("c")
        base = wid * fpw
        pltpu.sync_copy(idx_hbm.at[pl.ds(base, fpw)], idx_v)
        bufs = (buf0, buf1)
        sems = (sem0, sem1)

        def fire(c):
            return pltpu.async_copy(
                tab_hbm.at[idx_v.at[pl.ds(c * chunk, chunk)]],
                bufs[c % 2], sems[c % 2])

        pending = {0: fire(0)}
        for c in range(nch):
            if c + 1 < nch:
                pending[c + 1] = fire(c + 1)
            pending.pop(c).wait()
            pltpu.sync_copy(bufs[c % 2], out_hbm.at[pl.ds(base + c * chunk, chunk)])

    return k(tab, idx)


def _t1_dense(cl_sel, lwT, rwT, rb_row, W1, b1r, g1r, be1r, W2, b2r, g2r, be2r):
    def body(cl_ref, lw_ref, rw_ref, rb_ref, w1_ref, b1_ref, g1_ref, be1_ref,
             w2_ref, b2_ref, g2_ref, be2_ref, logit_ref, mu_ref):
        lat = cl_ref[:, 0:64]
        lib = cl_ref[:, 64:65]
        h = jnp.dot(lat, w1_ref[...], preferred_element_type=jnp.float32) + b1_ref[...]
        h = jnp.maximum(h, 0.0) * INV_SQ * g1_ref[...] + be1_ref[...]
        h = jnp.dot(h, w2_ref[...], preferred_element_type=jnp.float32) + b2_ref[...]
        h = jnp.maximum(h, 0.0) * INV_SQ * g2_ref[...] + be2_ref[...]
        logit_ref[...] = jnp.dot(h, lw_ref[...], preferred_element_type=jnp.float32)
        rho = jnp.dot(h, rw_ref[...], preferred_element_type=jnp.float32)
        mu_ref[...] = rb_ref[...] * jnp.exp(rho) * lib

    return pl.pallas_call(
        body,
        grid=(NGB,),
        in_specs=[
            pl.BlockSpec((B_CELLS, 128), lambda g: (0, 0)),
            pl.BlockSpec((32, GB * 32), lambda g: (0, g)),
            pl.BlockSpec((32, GB), lambda g: (0, g)),
            pl.BlockSpec((1, GB), lambda g: (0, g)),
            pl.BlockSpec((N_LATENT, N_HID), lambda g: (0, 0)),
            pl.BlockSpec((1, N_HID), lambda g: (0, 0)),
            pl.BlockSpec((1, N_HID), lambda g: (0, 0)),
            pl.BlockSpec((1, N_HID), lambda g: (0, 0)),
            pl.BlockSpec((N_HID, N_HID), lambda g: (0, 0)),
            pl.BlockSpec((1, N_HID), lambda g: (0, 0)),
            pl.BlockSpec((1, N_HID), lambda g: (0, 0)),
            pl.BlockSpec((1, N_HID), lambda g: (0, 0)),
        ],
        out_specs=[
            pl.BlockSpec((B_CELLS, GB * 32), lambda g: (0, g)),
            pl.BlockSpec((B_CELLS, GB), lambda g: (0, g)),
        ],
        out_shape=[
            jax.ShapeDtypeStruct((B_CELLS, GPAD * 32), jnp.float32),
            jax.ShapeDtypeStruct((B_CELLS, GPAD), jnp.float32),
        ],
    )(cl_sel, lwT, rwT, rb_row, W1, b1r, g1r, be1r, W2, b2r, g2r, be2r)


def _t2_mixture(delta, mix_g, x0c, x1c, selc):
    def body(d_ref, m_ref, x0_ref, x1_ref, sel_ref, out_ref):
        i = pl.program_id(0)
        raw = m_ref[...]
        loc = jax.nn.sigmoid(raw[:, 0:32])
        scale = SCALE_LB + jnp.exp(raw[:, 32:64])
        lsc = jnp.log(scale)
        dw = d_ref[...]
        sel = sel_ref[...]
        d = jnp.where(sel < 0.5, dw[:, 0:32],
            jnp.where(sel < 1.5, dw[:, 32:64],
            jnp.where(sel < 2.5, dw[:, 64:96], dw[:, 96:128])))
        lg = raw[:, 64:96] + d
        m = jnp.max(lg, axis=1, keepdims=True)
        lse = m + jnp.log(jnp.sum(jnp.exp(lg - m), axis=1, keepdims=True))
        log_mix = lg - lse

        def side(xcol):
            x = (xcol - WIN_A) / AB
            z = (x - loc) / scale
            t = log_mix + (-0.5 * z * z - lsc - 0.5 * LOG2PI)
            tm = jnp.max(t, axis=1, keepdims=True)
            return tm + jnp.log(jnp.sum(jnp.exp(t - tm), axis=1, keepdims=True))

        ll = side(x0_ref[...]) + side(x1_ref[...])
        fidx = i * FB + lax.broadcasted_iota(jnp.int32, (FB, 1), 0)
        ll = jnp.where(fidx < N_FRAG, ll, 0.0)

        @pl.when(i == 0)
        def _():
            out_ref[...] = jnp.zeros_like(out_ref)
        out_ref[...] += jnp.sum(ll).reshape(1, 1)

    return pl.pallas_call(
        body,
        grid=(NFB,),
        in_specs=[
            pl.BlockSpec((FB, 128), lambda i: (i, 0)),
            pl.BlockSpec((FB, 128), lambda i: (i, 0)),
            pl.BlockSpec((FB, 1), lambda i: (i, 0)),
            pl.BlockSpec((FB, 1), lambda i: (i, 0)),
            pl.BlockSpec((FB, 1), lambda i: (i, 0)),
        ],
        out_specs=pl.BlockSpec((1, 1), lambda i: (0, 0)),
        out_shape=jax.ShapeDtypeStruct((1, 1), jnp.float32),
    )(delta, mix_g, x0c, x1c, selc)


def _log_factorial(c):
    x = c + 1.0
    xs = jnp.maximum(x, 9.0)
    inv = 1.0 / xs
    stirling = ((xs - 0.5) * jnp.log(xs) - xs + 0.5 * LOG2PI
                + inv / 12.0 - (inv * inv * inv) / 360.0)
    tab = jnp.where(
        c < 0.5, 0.0,
        jnp.where(c < 1.5, 0.0,
        jnp.where(c < 2.5, 0.6931471805599453,
        jnp.where(c < 3.5, 1.791759469228055,
        jnp.where(c < 4.5, 3.1780538303479458,
        jnp.where(c < 5.5, 4.787491742782046,
        jnp.where(c < 6.5, 6.579251212010101,
                  8.525161361065415)))))))
    return jnp.where(c < 7.5, tab, stirling)


def _t3_poisson(cxg_row3, cxg_col, mu):
    def body(row_ref, col_ref, mu_ref, out_ref, cnt_ref):
        i = pl.program_id(0)

        @pl.when(i == 0)
        def _():
            cnt_ref[...] = jnp.zeros_like(cnt_ref)

        idx_row = row_ref[...].reshape(1, FB).astype(jnp.float32)
        cells_row = jnp.floor((idx_row + 0.5) / float(B_GENES))
        lane_f = lax.broadcasted_iota(jnp.int32, (B_CELLS, FB), 0).astype(jnp.float32)
        valid_row = (i * FB + lax.broadcasted_iota(jnp.int32, (B_CELLS, FB), 1)) < N_FRAG
        cell_ohT = jnp.where((lane_f == cells_row) & valid_row, 1.0, 0.0)

        idx_col = col_ref[...].astype(jnp.float32)
        cells_col = jnp.floor((idx_col + 0.5) / float(B_GENES))
        genes_col = idx_col - cells_col * float(B_GENES)
        gl = lax.broadcasted_iota(jnp.int32, (FB, B_GENES), 1).astype(jnp.float32)
        gene_oh = jnp.where(gl == genes_col, 1.0, 0.0)

        cnt_ref[...] += jnp.dot(cell_ohT.astype(jnp.bfloat16),
                                gene_oh.astype(jnp.bfloat16),
                                preferred_element_type=jnp.float32)

        @pl.when(i == NFB - 1)
        def _():
            cnt = cnt_ref[...]
            mu_v = mu_ref[...]
            pois = cnt * jnp.log(mu_v) - mu_v - _log_factorial(cnt)
            out_ref[...] = jnp.sum(pois).reshape(1, 1)

    return pl.pallas_call(
        body,
        grid=(NFB,),
        in_specs=[
            pl.BlockSpec((1, 1, FB), lambda i: (i, 0, 0)),
            pl.BlockSpec((FB, 1), lambda i: (i, 0)),
            pl.BlockSpec((B_CELLS, B_GENES), lambda i: (0, 0)),
        ],
        out_specs=pl.BlockSpec((1, 1), lambda i: (0, 0)),
        out_shape=jax.ShapeDtypeStruct((1, 1), jnp.float32),
        scratch_shapes=[pltpu.VMEM((B_CELLS, B_GENES), jnp.float32)],
    )(cxg_row3, cxg_col, mu)


def kernel(cells_oi, genes_oi, coordinates, local_gene_ix, local_cellxgene_ix,
           cell_latent_space, W1, b1, g1, be1, W2, b2, g2, be2,
           logit_w, rho_w, mloc, mscale, mlogit, libsize, rho_bias):
    f32 = jnp.float32
    i32 = jnp.int32

    lw_tab = logit_w.reshape(N_GENES, N_HID * N_COMP)
    gs_tab = jnp.concatenate([mloc, mscale, mlogit, rho_w], axis=1)
    rb_tab = jnp.concatenate(
        [rho_bias[:, None], jnp.zeros((N_GENES, 127), f32)], axis=1)
    cl_tab = jnp.concatenate(
        [cell_latent_space, libsize[:, None].astype(f32),
         jnp.zeros((N_CELLS, 63), f32)], axis=1)
    g_idx = jnp.concatenate(
        [genes_oi.astype(i32), jnp.zeros((GPAD - B_GENES,), i32)])
    c_idx = cells_oi.astype(i32)

    lw_sel, gs_sel, rb_sel, cl_sel = _sc_gather_tables(
        lw_tab, gs_tab, rb_tab, cl_tab, g_idx, c_idx)

    lwT = (lw_sel.reshape(GPAD, N_HID, N_COMP)
           .transpose(1, 0, 2).reshape(N_HID, GPAD * N_COMP))
    rwT = gs_sel[:, 96:128].T
    rb_row = rb_sel[:, 0].reshape(1, GPAD)
    mix_tab = gs_sel[:B_GENES]

    b1r = b1.reshape(1, N_HID); g1r = g1.reshape(1, N_HID); be1r = be1.reshape(1, N_HID)
    b2r = b2.reshape(1, N_HID); g2r = g2.reshape(1, N_HID); be2r = be2.reshape(1, N_HID)

    logit_flat2, mu_pad = _t1_dense(cl_sel, lwT, rwT, rb_row,
                                    W1, b1r, g1r, be1r, W2, b2r, g2r, be2r)
    logit_tab = logit_flat2.reshape(B_CELLS * GPAD * N_COMP // 128, 128)
    mu = mu_pad[:, :B_GENES]

    pad = FPAD - N_FRAG
    cxg = jnp.concatenate([local_cellxgene_ix.astype(i32), jnp.zeros((pad,), i32)])
    gix = jnp.concatenate([local_gene_ix.astype(i32), jnp.zeros((pad,), i32)])
    cxg2 = cxg + (GPAD - B_GENES) * (cxg // B_GENES)
    idx_wide = cxg2 // 4
    selc = (gix % 4).astype(f32).reshape(FPAD, 1)
    x0c = jnp.concatenate([coordinates[:, 0:1], jnp.zeros((pad, 1), f32)])
    x1c = jnp.concatenate([coordinates[:, 1:2], jnp.zeros((pad, 1), f32)])

    mix_g = _sc_gather_frag_rows(mix_tab, gix, "mix")
    delta = _sc_gather_frag_rows(logit_tab, idx_wide, "delta")

    ll = _t2_mixture(delta, mix_g, x0c, x1c, selc)

    cxg_row3 = cxg.reshape(NFB, 1, FB)
    cxg_col = cxg.reshape(FPAD, 1)
    pois = _t3_poisson(cxg_row3, cxg_col, mu)

    return -(ll[0, 0] + pois[0, 0])
